# baseline (speedup 1.0000x reference)
import functools

import jax
import jax.numpy as jnp
from jax import lax
from jax.experimental import pallas as pl
from jax.experimental.pallas import tpu as pltpu
from jax.experimental.pallas import tpu_sc as plsc


def _argmin_body(x_ref, w_ref, idx_ref):
    C, H, W = x_ref.shape[1:]
    X = x_ref[0].reshape(C, H * W)
    Wm = w_ref[...]
    N = H * W
    E, D = Wm.shape
    a = jnp.sum(X * X, axis=0, keepdims=True)
    w2 = jnp.sum(Wm * Wm, axis=1, keepdims=True)
    m2 = lax.dot_general(Wm + Wm, X, (((1,), (0,)), ((), ())),
                         preferred_element_type=jnp.float32)
    d = (w2 + a) - m2
    dmin = jnp.min(d, axis=0, keepdims=True)
    eidx = lax.broadcasted_iota(jnp.int32, (E, 1), 0).astype(jnp.float32)
    cand = jnp.where(d == dmin, eidx, jnp.float32(jnp.inf))
    idx_ref[0, 0] = jnp.min(cand, axis=0).astype(jnp.int32)


def _argmin_indices(x, embeddings):
    B, C, H, W = x.shape
    N = H * W
    E, D = embeddings.shape
    return pl.pallas_call(
        _argmin_body,
        grid=(B,),
        in_specs=[
            pl.BlockSpec((1, C, H, W), lambda b: (b, 0, 0, 0)),
            pl.BlockSpec((E, D), lambda b: (0, 0)),
        ],
        out_specs=pl.BlockSpec((1, 1, N), lambda b: (b, 0, 0)),
        out_shape=jax.ShapeDtypeStruct((B, 1, N), jnp.int32),
    )(x, embeddings)


def _sc_gather(table, idx_flat):
    E, D = table.shape
    (NB,) = idx_flat.shape
    info = plsc.get_sparse_core_info()
    NC, NS = info.num_cores, info.num_subcores
    NW = NC * NS
    b_per_w = NB // NW
    mesh = plsc.VectorSubcoreMesh(core_axis_name="c", subcore_axis_name="s")

    @functools.partial(
        pl.kernel,
        mesh=mesh,
        out_type=jax.ShapeDtypeStruct((NB, D), jnp.float32),
        scratch_types=[
            pltpu.VMEM((b_per_w,), jnp.int32),
            pltpu.VMEM((b_per_w, D), jnp.float32),
            pltpu.SemaphoreType.DMA,
        ],
        compiler_params=pltpu.CompilerParams(use_tc_tiling_on_sc=False),
    )
    def gather_k(table_hbm, idx_hbm, out_hbm, idx_v, rows_v, sem):
        wid = lax.axis_index("s") * NC + lax.axis_index("c")
        base = wid * b_per_w
        pltpu.sync_copy(idx_hbm.at[pl.ds(base, b_per_w)], idx_v)
        pltpu.async_copy(table_hbm.at[idx_v], rows_v, sem).wait()
        pltpu.sync_copy(rows_v, out_hbm.at[pl.ds(base, b_per_w)])

    return gather_k(table, idx_flat)


def kernel(input, embeddings):
    B, C, H, W = input.shape
    E, D = embeddings.shape
    N = H * W
    idx = _argmin_indices(input, embeddings)
    rows = _sc_gather(embeddings, idx.reshape(B * N))
    return rows.reshape(B, H, W, D).transpose(0, 3, 1, 2)


# speedup vs baseline: 1.0212x; 1.0212x over previous
import functools

import jax
import jax.numpy as jnp
from jax import lax
from jax.experimental import pallas as pl
from jax.experimental.pallas import tpu as pltpu
from jax.experimental.pallas import tpu_sc as plsc


def _argmin_body(x_ref, w_ref, idx_ref, wt_ref):
    X = x_ref[0]
    Wm = w_ref[...]
    C, N = X.shape
    E, D = Wm.shape
    a = jnp.sum(X * X, axis=0, keepdims=True)
    w2 = jnp.sum(Wm * Wm, axis=1, keepdims=True)
    m2 = lax.dot_general(Wm + Wm, X, (((1,), (0,)), ((), ())),
                         preferred_element_type=jnp.float32)
    d = (w2 + a) - m2
    dmin = jnp.min(d, axis=0, keepdims=True)
    eidx = lax.broadcasted_iota(jnp.int32, (E, 1), 0).astype(jnp.float32)
    cand = jnp.where(d == dmin, eidx, jnp.float32(jnp.inf))
    idx_ref[0, 0] = jnp.min(cand, axis=0).astype(jnp.int32)

    @pl.when(pl.program_id(0) == 0)
    def _():
        wt_ref[...] = lax.transpose(Wm, (1, 0))


def _argmin_indices(x, embeddings):
    B, C, N = x.shape
    E, D = embeddings.shape
    return pl.pallas_call(
        _argmin_body,
        grid=(B,),
        in_specs=[
            pl.BlockSpec((1, C, N), lambda b: (b, 0, 0)),
            pl.BlockSpec((E, D), lambda b: (0, 0)),
        ],
        out_specs=[
            pl.BlockSpec((1, 1, N), lambda b: (b, 0, 0)),
            pl.BlockSpec((D, E), lambda b: (0, 0)),
        ],
        out_shape=[
            jax.ShapeDtypeStruct((B, 1, N), jnp.int32),
            jax.ShapeDtypeStruct((D, E), jnp.float32),
        ],
    )(x, embeddings)


def _sc_lookup(wt, idx_flat, B, N):
    D, E = wt.shape
    (NB,) = idx_flat.shape
    info = plsc.get_sparse_core_info()
    NC, NS, L = info.num_cores, info.num_subcores, info.num_lanes
    NW = NC * NS
    cpw = D // NW
    mesh = plsc.VectorSubcoreMesh(core_axis_name="c", subcore_axis_name="s")

    @functools.partial(
        pl.kernel,
        mesh=mesh,
        out_type=jax.ShapeDtypeStruct((B, D, N), jnp.float32),
        scratch_types=[
            pltpu.VMEM((NB,), jnp.int32),
            pltpu.VMEM((E,), jnp.float32),
            pltpu.VMEM((E,), jnp.float32),
            pltpu.VMEM((B, N), jnp.float32),
            pltpu.VMEM((B, N), jnp.float32),
        ],
        compiler_params=pltpu.CompilerParams(use_tc_tiling_on_sc=False,
                                             needs_layout_passes=False),
    )
    def lookup_k(wt_hbm, idx_hbm, out_hbm, idx_v, wt0_v, wt1_v, o0_v, o1_v):
        wid = lax.axis_index("s") * NC + lax.axis_index("c")
        c0 = wid * cpw
        pltpu.sync_copy(idx_hbm, idx_v)
        pltpu.sync_copy(wt_hbm.at[c0], wt0_v)
        pltpu.sync_copy(wt_hbm.at[c0 + 1], wt1_v)

        def body(b, carry):
            for k in range(N // L):
                iv = idx_v[pl.ds(b * N + k * L, L)]
                o0_v[b, pl.ds(k * L, L)] = plsc.load_gather(wt0_v, [iv])
                o1_v[b, pl.ds(k * L, L)] = plsc.load_gather(wt1_v, [iv])
            return carry

        lax.fori_loop(0, B, body, 0)
        pltpu.sync_copy(o0_v, out_hbm.at[:, c0])
        pltpu.sync_copy(o1_v, out_hbm.at[:, c0 + 1])

    return lookup_k(wt, idx_flat)


def kernel(input, embeddings):
    B, C, H, W = input.shape
    E, D = embeddings.shape
    N = H * W
    x = input.reshape(B, C, N)
    idx, wt = _argmin_indices(x, embeddings)
    out = _sc_lookup(wt, idx.reshape(B * N), B, N)
    return out.reshape(B, D, H, W)


# speedup vs baseline: 1.2545x; 1.2285x over previous
import functools

import jax
import jax.numpy as jnp
from jax import lax
from jax.experimental import pallas as pl
from jax.experimental.pallas import tpu as pltpu
from jax.experimental.pallas import tpu_sc as plsc


def _argmin_body(x_ref, w_ref, idx_ref):
    G = x_ref.shape[0]
    Wm = w_ref[...]
    E, D = Wm.shape
    w2 = jnp.sum(Wm * Wm, axis=1, keepdims=True)
    Wm2 = Wm + Wm
    eidx = lax.broadcasted_iota(jnp.int32, (E, 1), 0).astype(jnp.float32)
    for g in range(G):
        X = x_ref[g]
        a = jnp.sum(X * X, axis=0, keepdims=True)
        m2 = lax.dot_general(Wm2, X, (((1,), (0,)), ((), ())),
                             preferred_element_type=jnp.float32)
        d = (w2 + a) - m2
        dmin = jnp.min(d, axis=0, keepdims=True)
        cand = jnp.where(d == dmin, eidx, jnp.float32(jnp.inf))
        idx_ref[g, 0] = jnp.min(cand, axis=0).astype(jnp.int32)


def _argmin_indices(x, embeddings, group=2):
    B, C, N = x.shape
    E, D = embeddings.shape
    G = group
    return pl.pallas_call(
        _argmin_body,
        grid=(B // G,),
        in_specs=[
            pl.BlockSpec((G, C, N), lambda b: (b, 0, 0)),
            pl.BlockSpec((E, D), lambda b: (0, 0)),
        ],
        out_specs=pl.BlockSpec((G, 1, N), lambda b: (b, 0, 0)),
        out_shape=jax.ShapeDtypeStruct((B, 1, N), jnp.int32),
    )(x, embeddings)


def _sc_gather(table, idx_flat):
    E, D = table.shape
    (NB,) = idx_flat.shape
    info = plsc.get_sparse_core_info()
    NC, NS = info.num_cores, info.num_subcores
    NW = NC * NS
    b_per_w = NB // NW
    mesh = plsc.VectorSubcoreMesh(core_axis_name="c", subcore_axis_name="s")

    @functools.partial(
        pl.kernel,
        mesh=mesh,
        out_type=jax.ShapeDtypeStruct((NB, D), jnp.float32),
        scratch_types=[
            pltpu.VMEM((b_per_w,), jnp.int32),
            pltpu.VMEM((b_per_w, D), jnp.float32),
            pltpu.SemaphoreType.DMA,
        ],
        compiler_params=pltpu.CompilerParams(use_tc_tiling_on_sc=False),
    )
    def gather_k(table_hbm, idx_hbm, out_hbm, idx_v, rows_v, sem):
        wid = lax.axis_index("s") * NC + lax.axis_index("c")
        base = wid * b_per_w
        pltpu.sync_copy(idx_hbm.at[pl.ds(base, b_per_w)], idx_v)
        pltpu.async_copy(table_hbm.at[idx_v], rows_v, sem).wait()
        pltpu.sync_copy(rows_v, out_hbm.at[pl.ds(base, b_per_w)])

    return gather_k(table, idx_flat)


def kernel(input, embeddings):
    B, C, H, W = input.shape
    E, D = embeddings.shape
    N = H * W
    x = input.reshape(B, C, N)
    idx = _argmin_indices(x, embeddings)
    rows = _sc_gather(embeddings, idx.reshape(B * N))
    return rows.reshape(B, H, W, D).transpose(0, 3, 1, 2)


# speedup vs baseline: 1.2810x; 1.0211x over previous
import functools

import jax
import jax.numpy as jnp
from jax import lax
from jax.experimental import pallas as pl
from jax.experimental.pallas import tpu as pltpu
from jax.experimental.pallas import tpu_sc as plsc


def _argmin_body(x_ref, w_ref, idx_ref):
    G = x_ref.shape[0]
    Wm = w_ref[...]
    E, D = Wm.shape
    w2 = jnp.sum(Wm * Wm, axis=1, keepdims=True)
    Wm2 = Wm + Wm
    eidx = lax.broadcasted_iota(jnp.int32, (E, 1), 0).astype(jnp.float32)
    for g in range(G):
        X = x_ref[g]
        a = jnp.sum(X * X, axis=0, keepdims=True)
        m2 = lax.dot_general(Wm2, X, (((1,), (0,)), ((), ())),
                             preferred_element_type=jnp.float32)
        d = (w2 + a) - m2
        dmin = jnp.min(d, axis=0, keepdims=True)
        cand = jnp.where(d == dmin, eidx, jnp.float32(jnp.inf))
        idx_ref[g, 0] = jnp.min(cand, axis=0).astype(jnp.int32)


def _argmin_indices(x, embeddings, group=4):
    B, C, N = x.shape
    E, D = embeddings.shape
    G = group
    return pl.pallas_call(
        _argmin_body,
        grid=(B // G,),
        in_specs=[
            pl.BlockSpec((G, C, N), lambda b: (b, 0, 0)),
            pl.BlockSpec((E, D), lambda b: (0, 0)),
        ],
        out_specs=pl.BlockSpec((G, 1, N), lambda b: (b, 0, 0)),
        out_shape=jax.ShapeDtypeStruct((B, 1, N), jnp.int32),
    )(x, embeddings)


def _sc_gather(table, idx_flat):
    E, D = table.shape
    (NB,) = idx_flat.shape
    info = plsc.get_sparse_core_info()
    NC, NS = info.num_cores, info.num_subcores
    NW = NC * NS
    b_per_w = NB // NW
    mesh = plsc.VectorSubcoreMesh(core_axis_name="c", subcore_axis_name="s")

    @functools.partial(
        pl.kernel,
        mesh=mesh,
        out_type=jax.ShapeDtypeStruct((NB, D), jnp.float32),
        scratch_types=[
            pltpu.VMEM((b_per_w,), jnp.int32),
            pltpu.VMEM((b_per_w, D), jnp.float32),
            pltpu.SemaphoreType.DMA,
        ],
        compiler_params=pltpu.CompilerParams(use_tc_tiling_on_sc=False),
    )
    def gather_k(table_hbm, idx_hbm, out_hbm, idx_v, rows_v, sem):
        wid = lax.axis_index("s") * NC + lax.axis_index("c")
        base = wid * b_per_w
        pltpu.sync_copy(idx_hbm.at[pl.ds(base, b_per_w)], idx_v)
        pltpu.async_copy(table_hbm.at[idx_v], rows_v, sem).wait()
        pltpu.sync_copy(rows_v, out_hbm.at[pl.ds(base, b_per_w)])

    return gather_k(table, idx_flat)


def kernel(input, embeddings):
    B, C, H, W = input.shape
    E, D = embeddings.shape
    N = H * W
    x = input.reshape(B, C, N)
    idx = _argmin_indices(x, embeddings)
    rows = _sc_gather(embeddings, idx.reshape(B * N))
    return rows.reshape(B, H, W, D).transpose(0, 3, 1, 2)
